# baseline (speedup 1.0000x reference)
import functools

import jax
import jax.numpy as jnp
from jax import lax
from jax.experimental import pallas as pl
from jax.experimental.pallas import tpu as pltpu
from jax.experimental.pallas import tpu_sc as plsc

N = 10000
D = 128
HD = D // 2
E = 320000
LAT = 64

NC, NS, L = 2, 16, 16
NW = NC * NS
EPT = E // NW
CH = 80
NCHUNK = EPT // CH
DRAIN_T = 10
DRAIN_R = N // DRAIN_T
ZR = 200

BR = 2000
GRID = N // BR

_MESH = plsc.VectorSubcoreMesh(
    core_axis_name="c", subcore_axis_name="s", num_cores=NC, num_subcores=NS)


@functools.partial(
    pl.kernel,
    out_type=jax.ShapeDtypeStruct((NC * N,), jnp.float32),
    mesh=_MESH,
    scratch_types=[
        pltpu.VMEM((NCHUNK, CH), jnp.int32),
        pltpu.VMEM((CH,), jnp.float32),
        pltpu.VMEM((DRAIN_R,), jnp.float32),
        pltpu.VMEM_SHARED((N,), jnp.float32),
        pltpu.SemaphoreType.DMA,
    ],
)
def _sc_degree(dst_hbm, zeros_hbm, out_hbm, idx_v, ones_v, stage_v, acc_sh,
               sem):
    c = lax.axis_index("c")
    s = lax.axis_index("s")
    wid = c * NS + s

    @pl.when(s < DRAIN_T)
    def _():
        pltpu.sync_copy(zeros_hbm, stage_v)
        pltpu.sync_copy(stage_v, acc_sh.at[pl.ds(s * DRAIN_R, DRAIN_R)])

    for k in range(CH // L):
        ones_v[pl.ds(k * L, L)] = jnp.full((L,), 1.0, jnp.float32)
    pltpu.sync_copy(dst_hbm.at[wid], idx_v)
    plsc.subcore_barrier()

    def chunk(j, carry):
        pltpu.sync_copy(ones_v, acc_sh.at[idx_v.at[j]], add=True)
        return carry

    lax.fori_loop(0, NCHUNK, chunk, 0)
    plsc.subcore_barrier()

    @pl.when(s < DRAIN_T)
    def _():
        pltpu.sync_copy(acc_sh.at[pl.ds(s * DRAIN_R, DRAIN_R)], stage_v)
        pltpu.sync_copy(stage_v,
                        out_hbm.at[pl.ds(c * N + s * DRAIN_R, DRAIN_R)])


NB = 5
NGRP = NCHUNK // NB


@functools.partial(
    pl.kernel,
    out_type=jax.ShapeDtypeStruct((NC, 2, N, HD), jnp.float32),
    mesh=_MESH,
    scratch_types=[
        pltpu.VMEM((NCHUNK, CH), jnp.int32),
        pltpu.VMEM((NCHUNK, CH), jnp.int32),
        pltpu.VMEM((NB, CH, HD), jnp.float32),
        pltpu.VMEM((NB, CH, HD), jnp.float32),
        pltpu.VMEM((ZR, HD), jnp.float32),
        pltpu.VMEM_SHARED((N, HD), jnp.float32),
        pltpu.SemaphoreType.DMA,
        pltpu.SemaphoreType.DMA,
        pltpu.SemaphoreType.DMA,
        pltpu.SemaphoreType.DMA,
    ],
    compiler_params=pltpu.CompilerParams(use_tc_tiling_on_sc=False),
)
def _sc_edge_pass(gsa_hbm, gsb_hbm, src_hbm, dst_hbm, zeros_hbm, out_hbm,
                  src_v, dst_v, buf_a, buf_b, stage_v, acc_sh, sem_a, sem_b,
                  sem_sa, sem_sb):
    c = lax.axis_index("c")
    s = lax.axis_index("s")
    wid = c * NS + s

    pltpu.sync_copy(src_hbm.at[wid], src_v)
    pltpu.sync_copy(dst_hbm.at[wid], dst_v)

    for h, gs_hbm in ((0, gsa_hbm), (1, gsb_hbm)):
        def fire(g, buf, sem):
            for b in range(NB):
                pltpu.async_copy(gs_hbm.at[src_v.at[g * NB + b]], buf.at[b],
                                 sem)

        def drain(buf, sem):
            for b in range(NB):
                pltpu.make_async_copy(gs_hbm.at[src_v.at[0]], buf.at[b],
                                      sem).wait()

        def fire_scatter(g, buf, sem):
            for b in range(NB):
                pltpu.async_copy(buf.at[b], acc_sh.at[dst_v.at[g * NB + b]],
                                 sem, add=True)

        def drain_scatter(buf, sem):
            for b in range(NB):
                pltpu.make_async_copy(buf.at[b], acc_sh.at[dst_v.at[0]],
                                      sem).wait()

        @pl.when(s < DRAIN_T)
        def _():
            pltpu.sync_copy(zeros_hbm, stage_v)
            for k in range(DRAIN_R // ZR):
                pltpu.sync_copy(
                    stage_v, acc_sh.at[pl.ds(s * DRAIN_R + k * ZR, ZR)])
        plsc.subcore_barrier()

        fire(0, buf_a, sem_a)
        fire(1, buf_b, sem_b)

        def grp(i, carry):
            g0 = 2 * i
            drain(buf_a, sem_a)
            fire_scatter(g0, buf_a, sem_sa)
            drain(buf_b, sem_b)
            fire_scatter(g0 + 1, buf_b, sem_sb)
            drain_scatter(buf_a, sem_sa)

            @pl.when(g0 + 2 < NGRP)
            def _():
                fire(g0 + 2, buf_a, sem_a)
            drain_scatter(buf_b, sem_sb)

            @pl.when(g0 + 3 < NGRP)
            def _():
                fire(g0 + 3, buf_b, sem_b)
            return carry

        lax.fori_loop(0, NGRP // 2, grp, 0)
        drain(buf_a, sem_a)
        fire_scatter(NGRP - 1, buf_a, sem_sa)
        drain_scatter(buf_a, sem_sa)
        plsc.subcore_barrier()

        @pl.when(s < DRAIN_T)
        def _():
            for k in range(DRAIN_R // ZR):
                sl = pl.ds(s * DRAIN_R + k * ZR, ZR)
                pltpu.sync_copy(acc_sh.at[sl], stage_v)
                pltpu.sync_copy(stage_v, out_hbm.at[c, h, sl])
        plsc.subcore_barrier()


def _pre_body(hc_ref, wh1_ref, bh1_ref, wh2_ref, bh2_ref, x_ref, w1_ref,
              out_ref):
    t = jnp.sum(wh1_ref[...] * hc_ref[...], axis=1)[None, :] + bh1_ref[...]
    t = jnp.maximum(t, 0.0)
    hom = jnp.sum(wh2_ref[...] * t, axis=1)[None, :] + bh2_ref[...]
    h = x_ref[...] + hom
    out_ref[...] = lax.dot_general(
        h, w1_ref[...], (((1,), (1,)), ((), ())),
        preferred_element_type=jnp.float32)


def _scale_body(degt_ref, p1_ref, gsa_ref, gsb_ref, dinv_ref):
    deg = 1.0 + degt_ref[...][:, 0:1] + degt_ref[...][:, 1:2]
    dinv = lax.rsqrt(deg)
    dinv_ref[...] = dinv
    gs = p1_ref[...] * dinv
    gsa_ref[...] = gs[:, :HD]
    gsb_ref[...] = gs[:, HD:]


def _mid_body(sp_ref, gsa_ref, gsb_ref, dinv_ref, b1_ref, w2_ref,
              gsa2_ref, gsb2_ref):
    gs1 = jnp.concatenate([gsa_ref[...], gsb_ref[...]], axis=1)
    ssum = jnp.concatenate([sp_ref[0, 0] + sp_ref[1, 0],
                            sp_ref[0, 1] + sp_ref[1, 1]], axis=1)
    h1 = jnp.maximum(dinv_ref[...] * (ssum + gs1) + b1_ref[...], 0.0)
    gs2 = lax.dot_general(
        h1, w2_ref[...], (((1,), (1,)), ((), ())),
        preferred_element_type=jnp.float32) * dinv_ref[...]
    gsa2_ref[...] = gs2[:, :HD]
    gsb2_ref[...] = gs2[:, HD:]


def _fin_body(sp_ref, gsa_ref, gsb_ref, dinv_ref, b2_ref, wmu_ref, bmu_ref,
              wlv_ref, blv_ref, mu_ref, lv_ref):
    gs2 = jnp.concatenate([gsa_ref[...], gsb_ref[...]], axis=1)
    ssum = jnp.concatenate([sp_ref[0, 0] + sp_ref[1, 0],
                            sp_ref[0, 1] + sp_ref[1, 1]], axis=1)
    h2 = jnp.maximum(dinv_ref[...] * (ssum + gs2) + b2_ref[...], 0.0)
    mu_ref[...] = lax.dot_general(
        h2, wmu_ref[...], (((1,), (1,)), ((), ())),
        preferred_element_type=jnp.float32) + bmu_ref[...]
    lv_ref[...] = lax.dot_general(
        h2, wlv_ref[...], (((1,), (1,)), ((), ())),
        preferred_element_type=jnp.float32) + blv_ref[...]


def _full(shape):
    nd = len(shape)
    return pl.BlockSpec(shape, lambda i, _nd=nd: (0,) * _nd)


def _rows(last):
    return pl.BlockSpec((BR, last), lambda i: (i, 0))


_SP_SPEC = pl.BlockSpec((NC, 2, BR, HD), lambda i: (0, 0, i, 0))


def kernel(x, edge_index, homophily_cond, Wh1, bh1, Wh2, bh2, W1, b1, W2, b2,
           Wmu, bmu, Wlv, blv):
    src = edge_index[0].reshape(NW, NCHUNK, CH)
    dst = edge_index[1].reshape(NW, NCHUNK, CH)
    zeros1 = jnp.zeros((DRAIN_R,), jnp.float32)
    zeros2 = jnp.zeros((ZR, HD), jnp.float32)
    hc = homophily_cond.reshape(1, 3)
    bh1r = bh1.reshape(1, 64)
    bh2r = bh2.reshape(1, D)
    b1r = b1.reshape(1, D)
    b2r = b2.reshape(1, D)
    bmur = bmu.reshape(1, LAT)
    blvr = blv.reshape(1, LAT)

    degp = _sc_degree(dst, zeros1).reshape(NC, N)
    degt = degp.T

    p1 = pl.pallas_call(
        _pre_body,
        grid=(GRID,),
        in_specs=[_full((1, 3)), _full((64, 3)), _full((1, 64)),
                  _full((D, 64)), _full((1, D)), _rows(D), _full((D, D))],
        out_specs=_rows(D),
        out_shape=jax.ShapeDtypeStruct((N, D), jnp.float32),
    )(hc, Wh1, bh1r, Wh2, bh2r, x, W1)

    gsa1, gsb1, dinv = pl.pallas_call(
        _scale_body,
        grid=(GRID,),
        in_specs=[pl.BlockSpec((BR, 2), lambda i: (i, 0)), _rows(D)],
        out_specs=[_rows(HD), _rows(HD), pl.BlockSpec((BR, 1), lambda i: (i, 0))],
        out_shape=[jax.ShapeDtypeStruct((N, HD), jnp.float32),
                   jax.ShapeDtypeStruct((N, HD), jnp.float32),
                   jax.ShapeDtypeStruct((N, 1), jnp.float32)],
    )(degt, p1)

    s1 = _sc_edge_pass(gsa1, gsb1, src, dst, zeros2)

    gsa2, gsb2 = pl.pallas_call(
        _mid_body,
        grid=(GRID,),
        in_specs=[_SP_SPEC, _rows(HD), _rows(HD),
                  pl.BlockSpec((BR, 1), lambda i: (i, 0)), _full((1, D)),
                  _full((D, D))],
        out_specs=[_rows(HD), _rows(HD)],
        out_shape=[jax.ShapeDtypeStruct((N, HD), jnp.float32),
                   jax.ShapeDtypeStruct((N, HD), jnp.float32)],
    )(s1, gsa1, gsb1, dinv, b1r, W2)

    s2 = _sc_edge_pass(gsa2, gsb2, src, dst, zeros2)

    mu, lv = pl.pallas_call(
        _fin_body,
        grid=(GRID,),
        in_specs=[_SP_SPEC, _rows(HD), _rows(HD),
                  pl.BlockSpec((BR, 1), lambda i: (i, 0)), _full((1, D)),
                  _full((LAT, D)), _full((1, LAT)), _full((LAT, D)),
                  _full((1, LAT))],
        out_specs=[_rows(LAT), _rows(LAT)],
        out_shape=[jax.ShapeDtypeStruct((N, LAT), jnp.float32),
                   jax.ShapeDtypeStruct((N, LAT), jnp.float32)],
    )(s2, gsa2, gsb2, dinv, b2r, Wmu, bmur, Wlv, blvr)

    return (mu, lv)


# speedup vs baseline: 1.1526x; 1.1526x over previous
import functools

import jax
import jax.numpy as jnp
from jax import lax
from jax.experimental import pallas as pl
from jax.experimental.pallas import tpu as pltpu
from jax.experimental.pallas import tpu_sc as plsc

N = 10000
D = 128
HD = D // 2
E = 320000
LAT = 64

NC, NS, L = 2, 16, 16
NW = NC * NS
EPT = E // NW
CH = 40
NCHUNK = EPT // CH
DRAIN_T = 10
DRAIN_R = N // DRAIN_T
ZR = 200

BR = 2000
GRID = N // BR

_MESH = plsc.VectorSubcoreMesh(
    core_axis_name="c", subcore_axis_name="s", num_cores=NC, num_subcores=NS)


@functools.partial(
    pl.kernel,
    out_type=jax.ShapeDtypeStruct((NC * N,), jnp.float32),
    mesh=_MESH,
    scratch_types=[
        pltpu.VMEM((NCHUNK, CH), jnp.int32),
        pltpu.VMEM((CH,), jnp.float32),
        pltpu.VMEM((DRAIN_R,), jnp.float32),
        pltpu.VMEM_SHARED((N,), jnp.float32),
        pltpu.SemaphoreType.DMA,
    ],
)
def _sc_degree(dst_hbm, zeros_hbm, out_hbm, idx_v, ones_v, stage_v, acc_sh,
               sem):
    c = lax.axis_index("c")
    s = lax.axis_index("s")
    wid = c * NS + s

    @pl.when(s < DRAIN_T)
    def _():
        pltpu.sync_copy(zeros_hbm, stage_v)
        pltpu.sync_copy(stage_v, acc_sh.at[pl.ds(s * DRAIN_R, DRAIN_R)])

    for k in range(CH // L):
        ones_v[pl.ds(k * L, L)] = jnp.full((L,), 1.0, jnp.float32)
    pltpu.sync_copy(dst_hbm.at[wid], idx_v)
    plsc.subcore_barrier()

    def chunk(j, carry):
        pltpu.sync_copy(ones_v, acc_sh.at[idx_v.at[j]], add=True)
        return carry

    lax.fori_loop(0, NCHUNK, chunk, 0)
    plsc.subcore_barrier()

    @pl.when(s < DRAIN_T)
    def _():
        pltpu.sync_copy(acc_sh.at[pl.ds(s * DRAIN_R, DRAIN_R)], stage_v)
        pltpu.sync_copy(stage_v,
                        out_hbm.at[pl.ds(c * N + s * DRAIN_R, DRAIN_R)])


NB = 5
NGRP = NCHUNK // NB
BKR = NB * CH


@functools.partial(
    pl.kernel,
    out_type=jax.ShapeDtypeStruct((NC, 2, N, HD), jnp.float32),
    mesh=_MESH,
    scratch_types=[
        pltpu.VMEM((NCHUNK, CH), jnp.int32),
        pltpu.VMEM((NCHUNK, CH), jnp.int32),
        pltpu.VMEM((BKR, HD), jnp.float32),
        pltpu.VMEM((BKR, HD), jnp.float32),
        pltpu.VMEM((BKR, HD), jnp.float32),
        pltpu.VMEM_SHARED((N, HD), jnp.float32),
        pltpu.SemaphoreType.DMA,
        pltpu.SemaphoreType.DMA,
        pltpu.SemaphoreType.DMA,
        pltpu.SemaphoreType.DMA,
        pltpu.SemaphoreType.DMA,
        pltpu.SemaphoreType.DMA,
    ],
    compiler_params=pltpu.CompilerParams(use_tc_tiling_on_sc=False),
)
def _sc_edge_pass(gsa_hbm, gsb_hbm, src_hbm, dst_hbm, zeros_hbm, out_hbm,
                  src_v, dst_v, buf_0, buf_1, buf_2, acc_sh,
                  sem_g0, sem_g1, sem_g2, sem_s0, sem_s1, sem_s2):
    c = lax.axis_index("c")
    s = lax.axis_index("s")
    wid = c * NS + s

    pltpu.sync_copy(src_hbm.at[wid], src_v)
    pltpu.sync_copy(dst_hbm.at[wid], dst_v)

    for h, gs_hbm in ((0, gsa_hbm), (1, gsb_hbm)):
        def fire(g, buf, sem):
            for b in range(NB):
                pltpu.async_copy(gs_hbm.at[src_v.at[g * NB + b]],
                                 buf.at[pl.ds(b * CH, CH)], sem)

        def drain(buf, sem):
            for b in range(NB):
                pltpu.make_async_copy(gs_hbm.at[src_v.at[0]],
                                      buf.at[pl.ds(b * CH, CH)], sem).wait()

        def fire_scatter(g, buf, sem):
            for b in range(NB):
                pltpu.async_copy(buf.at[pl.ds(b * CH, CH)],
                                 acc_sh.at[dst_v.at[g * NB + b]], sem,
                                 add=True)

        def drain_scatter(buf, sem):
            for b in range(NB):
                pltpu.make_async_copy(buf.at[pl.ds(b * CH, CH)],
                                      acc_sh.at[dst_v.at[0]], sem).wait()

        @pl.when(s < DRAIN_T)
        def _():
            pltpu.sync_copy(zeros_hbm, buf_0)
            for k in range(DRAIN_R // ZR):
                pltpu.sync_copy(
                    buf_0, acc_sh.at[pl.ds(s * DRAIN_R + k * ZR, ZR)])
        plsc.subcore_barrier()

        banks = (buf_0, buf_1, buf_2)
        sgs = (sem_g0, sem_g1, sem_g2)
        sss = (sem_s0, sem_s1, sem_s2)

        fire(0, buf_0, sem_g0)
        fire(1, buf_1, sem_g1)

        def period(g, k):
            bn = (k + 2) % 3
            drain(banks[k], sgs[k])
            fire_scatter(g, banks[k], sss[k])

            @pl.when((g + 2 < NGRP) & (g > 0))
            def _():
                drain_scatter(banks[bn], sss[bn])

            @pl.when(g + 2 < NGRP)
            def _():
                fire(g + 2, banks[bn], sgs[bn])

        def grp(i, carry):
            for k in range(3):
                period(3 * i + k, k)
            return carry

        lax.fori_loop(0, (NGRP - 2) // 3, grp, 0)
        period(NGRP - 2, (NGRP - 2) % 3)
        period(NGRP - 1, (NGRP - 1) % 3)
        drain_scatter(banks[(NGRP - 3) % 3], sss[(NGRP - 3) % 3])
        drain_scatter(banks[(NGRP - 2) % 3], sss[(NGRP - 2) % 3])
        drain_scatter(banks[(NGRP - 1) % 3], sss[(NGRP - 1) % 3])
        plsc.subcore_barrier()

        @pl.when(s < DRAIN_T)
        def _():
            for k in range(DRAIN_R // ZR):
                sl = pl.ds(s * DRAIN_R + k * ZR, ZR)
                pltpu.sync_copy(acc_sh.at[sl], buf_0)
                pltpu.sync_copy(buf_0, out_hbm.at[c, h, sl])
        plsc.subcore_barrier()


def _pre_body(hc_ref, wh1_ref, bh1_ref, wh2_ref, bh2_ref, x_ref, w1_ref,
              out_ref):
    t = jnp.sum(wh1_ref[...] * hc_ref[...], axis=1)[None, :] + bh1_ref[...]
    t = jnp.maximum(t, 0.0)
    hom = jnp.sum(wh2_ref[...] * t, axis=1)[None, :] + bh2_ref[...]
    h = x_ref[...] + hom
    out_ref[...] = lax.dot_general(
        h, w1_ref[...], (((1,), (1,)), ((), ())),
        preferred_element_type=jnp.float32)


def _scale_body(degt_ref, p1_ref, gsa_ref, gsb_ref, dinv_ref):
    deg = 1.0 + degt_ref[...][:, 0:1] + degt_ref[...][:, 1:2]
    dinv = lax.rsqrt(deg)
    dinv_ref[...] = dinv
    gs = p1_ref[...] * dinv
    gsa_ref[...] = gs[:, :HD]
    gsb_ref[...] = gs[:, HD:]


def _mid_body(sp_ref, gsa_ref, gsb_ref, dinv_ref, b1_ref, w2_ref,
              gsa2_ref, gsb2_ref):
    gs1 = jnp.concatenate([gsa_ref[...], gsb_ref[...]], axis=1)
    ssum = jnp.concatenate([sp_ref[0, 0] + sp_ref[1, 0],
                            sp_ref[0, 1] + sp_ref[1, 1]], axis=1)
    h1 = jnp.maximum(dinv_ref[...] * (ssum + gs1) + b1_ref[...], 0.0)
    gs2 = lax.dot_general(
        h1, w2_ref[...], (((1,), (1,)), ((), ())),
        preferred_element_type=jnp.float32) * dinv_ref[...]
    gsa2_ref[...] = gs2[:, :HD]
    gsb2_ref[...] = gs2[:, HD:]


def _fin_body(sp_ref, gsa_ref, gsb_ref, dinv_ref, b2_ref, wmu_ref, bmu_ref,
              wlv_ref, blv_ref, mu_ref, lv_ref):
    gs2 = jnp.concatenate([gsa_ref[...], gsb_ref[...]], axis=1)
    ssum = jnp.concatenate([sp_ref[0, 0] + sp_ref[1, 0],
                            sp_ref[0, 1] + sp_ref[1, 1]], axis=1)
    h2 = jnp.maximum(dinv_ref[...] * (ssum + gs2) + b2_ref[...], 0.0)
    mu_ref[...] = lax.dot_general(
        h2, wmu_ref[...], (((1,), (1,)), ((), ())),
        preferred_element_type=jnp.float32) + bmu_ref[...]
    lv_ref[...] = lax.dot_general(
        h2, wlv_ref[...], (((1,), (1,)), ((), ())),
        preferred_element_type=jnp.float32) + blv_ref[...]


def _full(shape):
    nd = len(shape)
    return pl.BlockSpec(shape, lambda i, _nd=nd: (0,) * _nd)


def _rows(last):
    return pl.BlockSpec((BR, last), lambda i: (i, 0))


_SP_SPEC = pl.BlockSpec((NC, 2, BR, HD), lambda i: (0, 0, i, 0))


def kernel(x, edge_index, homophily_cond, Wh1, bh1, Wh2, bh2, W1, b1, W2, b2,
           Wmu, bmu, Wlv, blv):
    src = edge_index[0].reshape(NW, NCHUNK, CH)
    dst = edge_index[1].reshape(NW, NCHUNK, CH)
    zeros1 = jnp.zeros((DRAIN_R,), jnp.float32)
    zeros2 = jnp.zeros((ZR, HD), jnp.float32)
    hc = homophily_cond.reshape(1, 3)
    bh1r = bh1.reshape(1, 64)
    bh2r = bh2.reshape(1, D)
    b1r = b1.reshape(1, D)
    b2r = b2.reshape(1, D)
    bmur = bmu.reshape(1, LAT)
    blvr = blv.reshape(1, LAT)

    degp = _sc_degree(dst, zeros1).reshape(NC, N)
    degt = degp.T

    p1 = pl.pallas_call(
        _pre_body,
        grid=(GRID,),
        in_specs=[_full((1, 3)), _full((64, 3)), _full((1, 64)),
                  _full((D, 64)), _full((1, D)), _rows(D), _full((D, D))],
        out_specs=_rows(D),
        out_shape=jax.ShapeDtypeStruct((N, D), jnp.float32),
    )(hc, Wh1, bh1r, Wh2, bh2r, x, W1)

    gsa1, gsb1, dinv = pl.pallas_call(
        _scale_body,
        grid=(GRID,),
        in_specs=[pl.BlockSpec((BR, 2), lambda i: (i, 0)), _rows(D)],
        out_specs=[_rows(HD), _rows(HD), pl.BlockSpec((BR, 1), lambda i: (i, 0))],
        out_shape=[jax.ShapeDtypeStruct((N, HD), jnp.float32),
                   jax.ShapeDtypeStruct((N, HD), jnp.float32),
                   jax.ShapeDtypeStruct((N, 1), jnp.float32)],
    )(degt, p1)

    s1 = _sc_edge_pass(gsa1, gsb1, src, dst, zeros2)

    gsa2, gsb2 = pl.pallas_call(
        _mid_body,
        grid=(GRID,),
        in_specs=[_SP_SPEC, _rows(HD), _rows(HD),
                  pl.BlockSpec((BR, 1), lambda i: (i, 0)), _full((1, D)),
                  _full((D, D))],
        out_specs=[_rows(HD), _rows(HD)],
        out_shape=[jax.ShapeDtypeStruct((N, HD), jnp.float32),
                   jax.ShapeDtypeStruct((N, HD), jnp.float32)],
    )(s1, gsa1, gsb1, dinv, b1r, W2)

    s2 = _sc_edge_pass(gsa2, gsb2, src, dst, zeros2)

    mu, lv = pl.pallas_call(
        _fin_body,
        grid=(GRID,),
        in_specs=[_SP_SPEC, _rows(HD), _rows(HD),
                  pl.BlockSpec((BR, 1), lambda i: (i, 0)), _full((1, D)),
                  _full((LAT, D)), _full((1, LAT)), _full((LAT, D)),
                  _full((1, LAT))],
        out_specs=[_rows(LAT), _rows(LAT)],
        out_shape=[jax.ShapeDtypeStruct((N, LAT), jnp.float32),
                   jax.ShapeDtypeStruct((N, LAT), jnp.float32)],
    )(s2, gsa2, gsb2, dinv, b2r, Wmu, bmur, Wlv, blvr)

    return (mu, lv)


# speedup vs baseline: 1.1756x; 1.0199x over previous
import functools

import jax
import jax.numpy as jnp
from jax import lax
from jax.experimental import pallas as pl
from jax.experimental.pallas import tpu as pltpu
from jax.experimental.pallas import tpu_sc as plsc

N = 10000
D = 128
HD = D // 2
E = 320000
LAT = 64

NC, NS, L = 2, 16, 16
NW = NC * NS
EPT = E // NW
CH = 80
NCHUNK = EPT // CH
DRAIN_T = 10
DRAIN_R = N // DRAIN_T
ZR = 200

BR = 2000
GRID = N // BR

_MESH = plsc.VectorSubcoreMesh(
    core_axis_name="c", subcore_axis_name="s", num_cores=NC, num_subcores=NS)


@functools.partial(
    pl.kernel,
    out_type=jax.ShapeDtypeStruct((NC * N,), jnp.float32),
    mesh=_MESH,
    scratch_types=[
        pltpu.VMEM((NCHUNK, CH), jnp.int32),
        pltpu.VMEM((CH,), jnp.float32),
        pltpu.VMEM((DRAIN_R,), jnp.float32),
        pltpu.VMEM_SHARED((N,), jnp.float32),
        pltpu.SemaphoreType.DMA,
    ],
)
def _sc_degree(dst_hbm, zeros_hbm, out_hbm, idx_v, ones_v, stage_v, acc_sh,
               sem):
    c = lax.axis_index("c")
    s = lax.axis_index("s")
    wid = c * NS + s

    @pl.when(s < DRAIN_T)
    def _():
        pltpu.sync_copy(zeros_hbm, stage_v)
        pltpu.sync_copy(stage_v, acc_sh.at[pl.ds(s * DRAIN_R, DRAIN_R)])

    for k in range(CH // L):
        ones_v[pl.ds(k * L, L)] = jnp.full((L,), 1.0, jnp.float32)
    pltpu.sync_copy(dst_hbm.at[wid], idx_v)
    plsc.subcore_barrier()

    def chunk(j, carry):
        pltpu.sync_copy(ones_v, acc_sh.at[idx_v.at[j]], add=True)
        return carry

    lax.fori_loop(0, NCHUNK, chunk, 0)
    plsc.subcore_barrier()

    @pl.when(s < DRAIN_T)
    def _():
        pltpu.sync_copy(acc_sh.at[pl.ds(s * DRAIN_R, DRAIN_R)], stage_v)
        pltpu.sync_copy(stage_v,
                        out_hbm.at[pl.ds(c * N + s * DRAIN_R, DRAIN_R)])


NB = 5
NGRP = NCHUNK // NB


@functools.partial(
    pl.kernel,
    out_type=jax.ShapeDtypeStruct((NC, 2, N, HD), jnp.float32),
    mesh=_MESH,
    scratch_types=[
        pltpu.VMEM((NCHUNK, CH), jnp.int32),
        pltpu.VMEM((NCHUNK, CH), jnp.int32),
        pltpu.VMEM((NB, CH, HD), jnp.float32),
        pltpu.VMEM((NB, CH, HD), jnp.float32),
        pltpu.VMEM((ZR, HD), jnp.float32),
        pltpu.VMEM_SHARED((N, HD), jnp.float32),
        pltpu.SemaphoreType.DMA,
        pltpu.SemaphoreType.DMA,
        pltpu.SemaphoreType.DMA,
        pltpu.SemaphoreType.DMA,
    ],
    compiler_params=pltpu.CompilerParams(use_tc_tiling_on_sc=False),
)
def _sc_edge_pass(gsa_hbm, gsb_hbm, src_hbm, dst_hbm, zeros_hbm, out_hbm,
                  src_v, dst_v, buf_a, buf_b, stage_v, acc_sh,
                  sem_a, sem_b, sem_sa, sem_sb):
    c = lax.axis_index("c")
    s = lax.axis_index("s")
    wid = c * NS + s

    pltpu.sync_copy(src_hbm.at[wid], src_v)
    pltpu.sync_copy(dst_hbm.at[wid], dst_v)

    for h, gs_hbm in ((0, gsa_hbm), (1, gsb_hbm)):
        def fire(g, buf, sem):
            for b in range(NB):
                pltpu.async_copy(gs_hbm.at[src_v.at[g * NB + b]], buf.at[b],
                                 sem)

        def drain(buf, sem):
            for b in range(NB):
                pltpu.make_async_copy(gs_hbm.at[src_v.at[0]], buf.at[b],
                                      sem).wait()

        def scatter(g, buf, sem):
            for b in range(NB):
                pltpu.async_copy(buf.at[b], acc_sh.at[dst_v.at[g * NB + b]],
                                 sem, add=True)
            for b in range(NB):
                pltpu.make_async_copy(buf.at[b], acc_sh.at[dst_v.at[0]],
                                      sem).wait()

        @pl.when(s < DRAIN_T)
        def _():
            pltpu.sync_copy(zeros_hbm, stage_v)
            for k in range(DRAIN_R // ZR):
                pltpu.sync_copy(
                    stage_v, acc_sh.at[pl.ds(s * DRAIN_R + k * ZR, ZR)])
        plsc.subcore_barrier()

        fire(0, buf_a, sem_a)
        fire(1, buf_b, sem_b)

        def grp(i, carry):
            g0 = 2 * i
            drain(buf_a, sem_a)
            scatter(g0, buf_a, sem_sa)

            @pl.when(g0 + 2 < NGRP)
            def _():
                fire(g0 + 2, buf_a, sem_a)
            drain(buf_b, sem_b)
            scatter(g0 + 1, buf_b, sem_sb)

            @pl.when(g0 + 3 < NGRP)
            def _():
                fire(g0 + 3, buf_b, sem_b)
            return carry

        lax.fori_loop(0, NGRP // 2, grp, 0)
        drain(buf_a, sem_a)
        scatter(NGRP - 1, buf_a, sem_sa)
        plsc.subcore_barrier()

        @pl.when(s < DRAIN_T)
        def _():
            for k in range(DRAIN_R // ZR):
                sl = pl.ds(s * DRAIN_R + k * ZR, ZR)
                pltpu.sync_copy(acc_sh.at[sl], stage_v)
                pltpu.sync_copy(stage_v, out_hbm.at[c, h, sl])
        plsc.subcore_barrier()


def _pre_body(hc_ref, wh1_ref, bh1_ref, wh2_ref, bh2_ref, x_ref, w1_ref,
              out_ref):
    t = jnp.sum(wh1_ref[...] * hc_ref[...], axis=1)[None, :] + bh1_ref[...]
    t = jnp.maximum(t, 0.0)
    hom = jnp.sum(wh2_ref[...] * t, axis=1)[None, :] + bh2_ref[...]
    h = x_ref[...] + hom
    out_ref[...] = lax.dot_general(
        h, w1_ref[...], (((1,), (1,)), ((), ())),
        preferred_element_type=jnp.float32)


def _scale_body(degt_ref, p1_ref, gsa_ref, gsb_ref, dinv_ref):
    deg = 1.0 + degt_ref[...][:, 0:1] + degt_ref[...][:, 1:2]
    dinv = lax.rsqrt(deg)
    dinv_ref[...] = dinv
    gs = p1_ref[...] * dinv
    gsa_ref[...] = gs[:, :HD]
    gsb_ref[...] = gs[:, HD:]


def _mid_body(sp_ref, gsa_ref, gsb_ref, dinv_ref, b1_ref, w2_ref,
              gsa2_ref, gsb2_ref):
    gs1 = jnp.concatenate([gsa_ref[...], gsb_ref[...]], axis=1)
    ssum = jnp.concatenate([sp_ref[0, 0] + sp_ref[1, 0],
                            sp_ref[0, 1] + sp_ref[1, 1]], axis=1)
    h1 = jnp.maximum(dinv_ref[...] * (ssum + gs1) + b1_ref[...], 0.0)
    gs2 = lax.dot_general(
        h1, w2_ref[...], (((1,), (1,)), ((), ())),
        preferred_element_type=jnp.float32) * dinv_ref[...]
    gsa2_ref[...] = gs2[:, :HD]
    gsb2_ref[...] = gs2[:, HD:]


def _fin_body(sp_ref, gsa_ref, gsb_ref, dinv_ref, b2_ref, wmu_ref, bmu_ref,
              wlv_ref, blv_ref, mu_ref, lv_ref):
    gs2 = jnp.concatenate([gsa_ref[...], gsb_ref[...]], axis=1)
    ssum = jnp.concatenate([sp_ref[0, 0] + sp_ref[1, 0],
                            sp_ref[0, 1] + sp_ref[1, 1]], axis=1)
    h2 = jnp.maximum(dinv_ref[...] * (ssum + gs2) + b2_ref[...], 0.0)
    mu_ref[...] = lax.dot_general(
        h2, wmu_ref[...], (((1,), (1,)), ((), ())),
        preferred_element_type=jnp.float32) + bmu_ref[...]
    lv_ref[...] = lax.dot_general(
        h2, wlv_ref[...], (((1,), (1,)), ((), ())),
        preferred_element_type=jnp.float32) + blv_ref[...]


def _full(shape):
    nd = len(shape)
    return pl.BlockSpec(shape, lambda i, _nd=nd: (0,) * _nd)


def _rows(last):
    return pl.BlockSpec((BR, last), lambda i: (i, 0))


_SP_SPEC = pl.BlockSpec((NC, 2, BR, HD), lambda i: (0, 0, i, 0))


def kernel(x, edge_index, homophily_cond, Wh1, bh1, Wh2, bh2, W1, b1, W2, b2,
           Wmu, bmu, Wlv, blv):
    src = edge_index[0].reshape(NW, NCHUNK, CH)
    dst = edge_index[1].reshape(NW, NCHUNK, CH)
    zeros1 = jnp.zeros((DRAIN_R,), jnp.float32)
    zeros2 = jnp.zeros((ZR, HD), jnp.float32)
    hc = homophily_cond.reshape(1, 3)
    bh1r = bh1.reshape(1, 64)
    bh2r = bh2.reshape(1, D)
    b1r = b1.reshape(1, D)
    b2r = b2.reshape(1, D)
    bmur = bmu.reshape(1, LAT)
    blvr = blv.reshape(1, LAT)

    degp = _sc_degree(dst, zeros1).reshape(NC, N)
    degt = degp.T

    p1 = pl.pallas_call(
        _pre_body,
        grid=(GRID,),
        in_specs=[_full((1, 3)), _full((64, 3)), _full((1, 64)),
                  _full((D, 64)), _full((1, D)), _rows(D), _full((D, D))],
        out_specs=_rows(D),
        out_shape=jax.ShapeDtypeStruct((N, D), jnp.float32),
    )(hc, Wh1, bh1r, Wh2, bh2r, x, W1)

    gsa1, gsb1, dinv = pl.pallas_call(
        _scale_body,
        grid=(GRID,),
        in_specs=[pl.BlockSpec((BR, 2), lambda i: (i, 0)), _rows(D)],
        out_specs=[_rows(HD), _rows(HD), pl.BlockSpec((BR, 1), lambda i: (i, 0))],
        out_shape=[jax.ShapeDtypeStruct((N, HD), jnp.float32),
                   jax.ShapeDtypeStruct((N, HD), jnp.float32),
                   jax.ShapeDtypeStruct((N, 1), jnp.float32)],
    )(degt, p1)

    s1 = _sc_edge_pass(gsa1, gsb1, src, dst, zeros2)

    gsa2, gsb2 = pl.pallas_call(
        _mid_body,
        grid=(GRID,),
        in_specs=[_SP_SPEC, _rows(HD), _rows(HD),
                  pl.BlockSpec((BR, 1), lambda i: (i, 0)), _full((1, D)),
                  _full((D, D))],
        out_specs=[_rows(HD), _rows(HD)],
        out_shape=[jax.ShapeDtypeStruct((N, HD), jnp.float32),
                   jax.ShapeDtypeStruct((N, HD), jnp.float32)],
    )(s1, gsa1, gsb1, dinv, b1r, W2)

    s2 = _sc_edge_pass(gsa2, gsb2, src, dst, zeros2)

    mu, lv = pl.pallas_call(
        _fin_body,
        grid=(GRID,),
        in_specs=[_SP_SPEC, _rows(HD), _rows(HD),
                  pl.BlockSpec((BR, 1), lambda i: (i, 0)), _full((1, D)),
                  _full((LAT, D)), _full((1, LAT)), _full((LAT, D)),
                  _full((1, LAT))],
        out_specs=[_rows(LAT), _rows(LAT)],
        out_shape=[jax.ShapeDtypeStruct((N, LAT), jnp.float32),
                   jax.ShapeDtypeStruct((N, LAT), jnp.float32)],
    )(s2, gsa2, gsb2, dinv, b2r, Wmu, bmur, Wlv, blvr)

    return (mu, lv)


# speedup vs baseline: 1.3121x; 1.1162x over previous
import functools

import jax
import jax.numpy as jnp
from jax import lax
from jax.experimental import pallas as pl
from jax.experimental.pallas import tpu as pltpu
from jax.experimental.pallas import tpu_sc as plsc

N = 10000
D = 128
HD = D // 2
E = 320000
LAT = 64

NC, NS, L = 2, 16, 16
NW = NC * NS
EPT = E // NW
CH = 80
NCHUNK = EPT // CH
DRAIN_T = 10
DRAIN_R = N // DRAIN_T
ZR = 200

BR = 2000
GRID = N // BR

_MESH = plsc.VectorSubcoreMesh(
    core_axis_name="c", subcore_axis_name="s", num_cores=NC, num_subcores=NS)


@functools.partial(
    pl.kernel,
    out_type=jax.ShapeDtypeStruct((NC * N,), jnp.float32),
    mesh=_MESH,
    scratch_types=[
        pltpu.VMEM((NCHUNK, CH), jnp.int32),
        pltpu.VMEM((CH,), jnp.float32),
        pltpu.VMEM((DRAIN_R,), jnp.float32),
        pltpu.VMEM_SHARED((N,), jnp.float32),
        pltpu.SemaphoreType.DMA,
    ],
)
def _sc_degree(dst_hbm, zeros_hbm, out_hbm, idx_v, ones_v, stage_v, acc_sh,
               sem):
    c = lax.axis_index("c")
    s = lax.axis_index("s")
    wid = c * NS + s

    @pl.when(s < DRAIN_T)
    def _():
        pltpu.sync_copy(zeros_hbm, stage_v)
        pltpu.sync_copy(stage_v, acc_sh.at[pl.ds(s * DRAIN_R, DRAIN_R)])

    for k in range(CH // L):
        ones_v[pl.ds(k * L, L)] = jnp.full((L,), 1.0, jnp.float32)
    pltpu.sync_copy(dst_hbm.at[wid], idx_v)
    plsc.subcore_barrier()

    def chunk(j, carry):
        pltpu.sync_copy(ones_v, acc_sh.at[idx_v.at[j]], add=True)
        return carry

    lax.fori_loop(0, NCHUNK, chunk, 0)
    plsc.subcore_barrier()

    @pl.when(s < DRAIN_T)
    def _():
        pltpu.sync_copy(acc_sh.at[pl.ds(s * DRAIN_R, DRAIN_R)], stage_v)
        pltpu.sync_copy(stage_v,
                        out_hbm.at[pl.ds(c * N + s * DRAIN_R, DRAIN_R)])


NB = 5
NGRP = NCHUNK // NB


@functools.partial(
    pl.kernel,
    out_type=jax.ShapeDtypeStruct((NC, N, D), jnp.float32),
    mesh=_MESH,
    scratch_types=[
        pltpu.VMEM((NCHUNK, CH), jnp.int32),
        pltpu.VMEM((NCHUNK, CH), jnp.int32),
        pltpu.VMEM((NB, CH, HD), jnp.float32),
        pltpu.VMEM((NB, CH, HD), jnp.float32),
        pltpu.VMEM((ZR, HD), jnp.float32),
        pltpu.VMEM_SHARED((N, HD), jnp.float32),
        pltpu.SemaphoreType.DMA,
        pltpu.SemaphoreType.DMA,
        pltpu.SemaphoreType.DMA,
        pltpu.SemaphoreType.DMA,
    ],
    compiler_params=pltpu.CompilerParams(use_tc_tiling_on_sc=False),
)
def _sc_edge_pass(gsa_hbm, gsb_hbm, src_hbm, dst_hbm, zeros_hbm, out_hbm,
                  src_v, dst_v, buf_a, buf_b, stage_v, acc_sh,
                  sem_a, sem_b, sem_sa, sem_sb):
    c = lax.axis_index("c")
    s = lax.axis_index("s")
    wid = c * NS + s

    pltpu.sync_copy(src_hbm.at[wid], src_v)
    pltpu.sync_copy(dst_hbm.at[wid], dst_v)

    for h, gs_hbm in ((0, gsa_hbm), (1, gsb_hbm)):
        def fire(g, buf, sem):
            for b in range(NB):
                pltpu.async_copy(gs_hbm.at[src_v.at[g * NB + b]], buf.at[b],
                                 sem)

        def drain(buf, sem):
            for b in range(NB):
                pltpu.make_async_copy(gs_hbm.at[src_v.at[0]], buf.at[b],
                                      sem).wait()

        def scatter(g, buf, sem):
            for b in range(NB):
                pltpu.async_copy(buf.at[b], acc_sh.at[dst_v.at[g * NB + b]],
                                 sem, add=True)
            for b in range(NB):
                pltpu.make_async_copy(buf.at[b], acc_sh.at[dst_v.at[0]],
                                      sem).wait()

        @pl.when(s < DRAIN_T)
        def _():
            pltpu.sync_copy(zeros_hbm, stage_v)
            for k in range(DRAIN_R // ZR):
                pltpu.sync_copy(
                    stage_v, acc_sh.at[pl.ds(s * DRAIN_R + k * ZR, ZR)])
        plsc.subcore_barrier()

        fire(0, buf_a, sem_a)
        fire(1, buf_b, sem_b)

        def grp(i, carry):
            g0 = 2 * i
            drain(buf_a, sem_a)
            scatter(g0, buf_a, sem_sa)

            @pl.when(g0 + 2 < NGRP)
            def _():
                fire(g0 + 2, buf_a, sem_a)
            drain(buf_b, sem_b)
            scatter(g0 + 1, buf_b, sem_sb)

            @pl.when(g0 + 3 < NGRP)
            def _():
                fire(g0 + 3, buf_b, sem_b)
            return carry

        lax.fori_loop(0, NGRP // 2, grp, 0)
        drain(buf_a, sem_a)
        scatter(NGRP - 1, buf_a, sem_sa)
        plsc.subcore_barrier()

        @pl.when(s < DRAIN_T)
        def _():
            for k in range(DRAIN_R // ZR):
                sl = pl.ds(s * DRAIN_R + k * ZR, ZR)
                pltpu.sync_copy(acc_sh.at[sl], stage_v)
                pltpu.sync_copy(stage_v,
                                out_hbm.at[c, sl, pl.ds(h * HD, HD)])
        plsc.subcore_barrier()


def _pre_body(hc_ref, wh1_ref, bh1_ref, wh2_ref, bh2_ref, x_ref, w1_ref,
              out_ref):
    t = jnp.sum(wh1_ref[...] * hc_ref[...], axis=1)[None, :] + bh1_ref[...]
    t = jnp.maximum(t, 0.0)
    hom = jnp.sum(wh2_ref[...] * t, axis=1)[None, :] + bh2_ref[...]
    h = x_ref[...] + hom
    out_ref[...] = lax.dot_general(
        h, w1_ref[...], (((1,), (1,)), ((), ())),
        preferred_element_type=jnp.float32)


def _scale_body(degt_ref, p1_ref, gsa_ref, gsb_ref, dinv_ref):
    deg = 1.0 + degt_ref[...][:, 0:1] + degt_ref[...][:, 1:2]
    dinv = lax.rsqrt(deg)
    dinv_ref[...] = dinv
    gs = p1_ref[...] * dinv
    gsa_ref[...] = gs[:, :HD]
    gsb_ref[...] = gs[:, HD:]


def _mid_body(sp_ref, gsa_ref, gsb_ref, dinv_ref, b1_ref, w2_ref,
              gsa2_ref, gsb2_ref):
    gs1 = jnp.concatenate([gsa_ref[...], gsb_ref[...]], axis=1)
    ssum = sp_ref[0] + sp_ref[1]
    h1 = jnp.maximum(dinv_ref[...] * (ssum + gs1) + b1_ref[...], 0.0)
    gs2 = lax.dot_general(
        h1, w2_ref[...], (((1,), (1,)), ((), ())),
        preferred_element_type=jnp.float32) * dinv_ref[...]
    gsa2_ref[...] = gs2[:, :HD]
    gsb2_ref[...] = gs2[:, HD:]


def _fin_body(sp_ref, gsa_ref, gsb_ref, dinv_ref, b2_ref, wmu_ref, bmu_ref,
              wlv_ref, blv_ref, mu_ref, lv_ref):
    gs2 = jnp.concatenate([gsa_ref[...], gsb_ref[...]], axis=1)
    ssum = sp_ref[0] + sp_ref[1]
    h2 = jnp.maximum(dinv_ref[...] * (ssum + gs2) + b2_ref[...], 0.0)
    mu_ref[...] = lax.dot_general(
        h2, wmu_ref[...], (((1,), (1,)), ((), ())),
        preferred_element_type=jnp.float32) + bmu_ref[...]
    lv_ref[...] = lax.dot_general(
        h2, wlv_ref[...], (((1,), (1,)), ((), ())),
        preferred_element_type=jnp.float32) + blv_ref[...]


def _full(shape):
    nd = len(shape)
    return pl.BlockSpec(shape, lambda i, _nd=nd: (0,) * _nd)


def _rows(last):
    return pl.BlockSpec((BR, last), lambda i: (i, 0))


_SP_SPEC = pl.BlockSpec((NC, BR, D), lambda i: (0, i, 0))


def kernel(x, edge_index, homophily_cond, Wh1, bh1, Wh2, bh2, W1, b1, W2, b2,
           Wmu, bmu, Wlv, blv):
    src = edge_index[0].reshape(NW, NCHUNK, CH)
    dst = edge_index[1].reshape(NW, NCHUNK, CH)
    zeros1 = jnp.zeros((DRAIN_R,), jnp.float32)
    zeros2 = jnp.zeros((ZR, HD), jnp.float32)
    hc = homophily_cond.reshape(1, 3)
    bh1r = bh1.reshape(1, 64)
    bh2r = bh2.reshape(1, D)
    b1r = b1.reshape(1, D)
    b2r = b2.reshape(1, D)
    bmur = bmu.reshape(1, LAT)
    blvr = blv.reshape(1, LAT)

    degp = _sc_degree(dst, zeros1).reshape(NC, N)
    degt = degp.T

    p1 = pl.pallas_call(
        _pre_body,
        grid=(GRID,),
        in_specs=[_full((1, 3)), _full((64, 3)), _full((1, 64)),
                  _full((D, 64)), _full((1, D)), _rows(D), _full((D, D))],
        out_specs=_rows(D),
        out_shape=jax.ShapeDtypeStruct((N, D), jnp.float32),
    )(hc, Wh1, bh1r, Wh2, bh2r, x, W1)

    gsa1, gsb1, dinv = pl.pallas_call(
        _scale_body,
        grid=(GRID,),
        in_specs=[pl.BlockSpec((BR, 2), lambda i: (i, 0)), _rows(D)],
        out_specs=[_rows(HD), _rows(HD), pl.BlockSpec((BR, 1), lambda i: (i, 0))],
        out_shape=[jax.ShapeDtypeStruct((N, HD), jnp.float32),
                   jax.ShapeDtypeStruct((N, HD), jnp.float32),
                   jax.ShapeDtypeStruct((N, 1), jnp.float32)],
    )(degt, p1)

    s1 = _sc_edge_pass(gsa1, gsb1, src, dst, zeros2)

    gsa2, gsb2 = pl.pallas_call(
        _mid_body,
        grid=(GRID,),
        in_specs=[_SP_SPEC, _rows(HD), _rows(HD),
                  pl.BlockSpec((BR, 1), lambda i: (i, 0)), _full((1, D)),
                  _full((D, D))],
        out_specs=[_rows(HD), _rows(HD)],
        out_shape=[jax.ShapeDtypeStruct((N, HD), jnp.float32),
                   jax.ShapeDtypeStruct((N, HD), jnp.float32)],
    )(s1, gsa1, gsb1, dinv, b1r, W2)

    s2 = _sc_edge_pass(gsa2, gsb2, src, dst, zeros2)

    mu, lv = pl.pallas_call(
        _fin_body,
        grid=(GRID,),
        in_specs=[_SP_SPEC, _rows(HD), _rows(HD),
                  pl.BlockSpec((BR, 1), lambda i: (i, 0)), _full((1, D)),
                  _full((LAT, D)), _full((1, LAT)), _full((LAT, D)),
                  _full((1, LAT))],
        out_specs=[_rows(LAT), _rows(LAT)],
        out_shape=[jax.ShapeDtypeStruct((N, LAT), jnp.float32),
                   jax.ShapeDtypeStruct((N, LAT), jnp.float32)],
    )(s2, gsa2, gsb2, dinv, b2r, Wmu, bmur, Wlv, blvr)

    return (mu, lv)
